# baseline (speedup 1.0000x reference)
import jax
import jax.numpy as jnp
from jax import lax
from jax.experimental import pallas as pl
from jax.experimental.pallas import tpu as pltpu
from jax.experimental.pallas import tpu_sc as plsc

N = 10000
D = 128
EPS = 1e-5
NC = 2
NS = 16
CH = 64

NCHUNK = 2500
CPAD = 2560
CPT = 160
LAST_CT = NCHUNK - (NS - 1) * CPT
HALF = CPT // 4
NBUF = 4
PF = 3

ROW_CHUNK = 632
ROW_LAST = N - (NS - 1) * ROW_CHUNK
ELEM_CHUNK = 640
ELEM_LAST = N - (NS - 1) * ELEM_CHUNK


def _sc_mesh():
    return plsc.VectorSubcoreMesh(
        core_axis_name="c", subcore_axis_name="s", num_cores=NC, num_subcores=NS
    )


def _degree_body(col_hbm, out_hbm, idxc_all, onesv, bufv, sem, deg_spm):
    c = lax.axis_index("c")
    s = lax.axis_index("s")
    nct = jnp.where(s == NS - 1, LAST_CT, CPT)

    def fill(i, _):
        bufv[pl.ds(i * 16, 16)] = jnp.zeros((16,), jnp.float32)
        return 0

    lax.fori_loop(0, ELEM_CHUNK // 16, fill, 0)

    def fill1(i, _):
        onesv[pl.ds(i * 16, 16)] = jnp.ones((16,), jnp.float32)
        return 0

    lax.fori_loop(0, CH // 16, fill1, 0)

    pltpu.sync_copy(col_hbm.at[pl.ds(CPAD * c + CPT * s, CPT)], idxc_all)

    @pl.when(s < NS - 1)
    def _():
        pltpu.sync_copy(bufv, deg_spm.at[pl.ds(s * ELEM_CHUNK, ELEM_CHUNK)])

    @pl.when(s == NS - 1)
    def _():
        pltpu.sync_copy(
            bufv.at[pl.ds(0, ELEM_LAST)],
            deg_spm.at[pl.ds((NS - 1) * ELEM_CHUNK, ELEM_LAST)],
        )

    plsc.subcore_barrier()

    FIRE = 8

    def step(k8, _):
        for j in range(FIRE):
            kk = FIRE * k8 + j

            @pl.when(kk < nct)
            def _():
                pltpu.async_copy(
                    onesv, deg_spm.at[idxc_all.at[kk]], sem, add=True
                )

        for j in range(FIRE):
            kk = FIRE * k8 + j

            @pl.when(kk < nct)
            def _():
                pltpu.make_async_copy(
                    onesv, deg_spm.at[idxc_all.at[kk]], sem
                ).wait()

        return 0

    lax.fori_loop(0, CPT // FIRE, step, 0)
    plsc.subcore_barrier()

    @pl.when(s < NS - 1)
    def _():
        pltpu.sync_copy(deg_spm.at[pl.ds(s * ELEM_CHUNK, ELEM_CHUNK)], bufv)
        pltpu.sync_copy(
            bufv, out_hbm.at[pl.ds(c * N + s * ELEM_CHUNK, ELEM_CHUNK)]
        )

    @pl.when(s == NS - 1)
    def _():
        pltpu.sync_copy(
            deg_spm.at[pl.ds((NS - 1) * ELEM_CHUNK, ELEM_LAST)],
            bufv.at[pl.ds(0, ELEM_LAST)],
        )
        pltpu.sync_copy(
            bufv.at[pl.ds(0, ELEM_LAST)],
            out_hbm.at[pl.ds(c * N + (NS - 1) * ELEM_CHUNK, ELEM_LAST)],
        )


def _sc_degree(col2):
    return pl.kernel(
        _degree_body,
        out_type=jax.ShapeDtypeStruct((NC * N,), jnp.float32),
        mesh=_sc_mesh(),
        scratch_types=[
            pltpu.VMEM((CPT, CH), jnp.int32),
            pltpu.VMEM((CH,), jnp.float32),
            pltpu.VMEM((ELEM_CHUNK,), jnp.float32),
            pltpu.SemaphoreType.DMA,
            pltpu.VMEM_SHARED((N,), jnp.float32),
        ],
    )(col2)


def _aggregate_body(hs_hbm, row_hbm, col_hbm, out_hbm,
                    idxr_h, idxc_h, rowsv, sem_g, sem_s, sem_i, acc_spm):
    c = lax.axis_index("c")
    s = lax.axis_index("s")
    nct = jnp.where(s == NS - 1, LAST_CT, CPT)
    ibase = CPAD * c + CPT * s
    ZB = NBUF - 1

    def _fetch_idx(g0):
        pltpu.async_copy(row_hbm.at[pl.ds(ibase + g0, HALF)], idxr_h, sem_i)
        pltpu.async_copy(col_hbm.at[pl.ds(ibase + g0, HALF)], idxc_h, sem_i)

    def _wait_idx(g0):
        pltpu.make_async_copy(
            row_hbm.at[pl.ds(ibase + g0, HALF)], idxr_h, sem_i
        ).wait()
        pltpu.make_async_copy(
            col_hbm.at[pl.ds(ibase + g0, HALF)], idxc_h, sem_i
        ).wait()

    _fetch_idx(0)

    def fill(i, _):
        rowsv[ZB, i // 8, pl.ds((i % 8) * 16, 16)] = jnp.zeros((16,), jnp.float32)
        return 0

    lax.fori_loop(0, CH * D // 16, fill, 0)

    r0 = s * ROW_CHUNK

    def _gather(kk_local, b):
        pltpu.async_copy(
            hs_hbm.at[idxr_h.at[kk_local]], rowsv.at[b], sem_g.at[b]
        )

    def _wait_gather(kk_local, b):
        pltpu.make_async_copy(
            hs_hbm.at[idxr_h.at[kk_local]], rowsv.at[b], sem_g.at[b]
        ).wait()

    def _scatter(kk_local, b):
        pltpu.async_copy(
            rowsv.at[b], acc_spm.at[idxc_h.at[kk_local]], sem_s.at[b],
            add=True,
        )

    def _wait_scatter(kk_local, b):
        pltpu.make_async_copy(
            rowsv.at[b], acc_spm.at[idxc_h.at[kk_local]], sem_s.at[b]
        ).wait()

    _wait_idx(0)
    for j in range(PF):
        _gather(j, j % NBUF)

    def _zero_rows(nrows):
        nf = nrows // CH
        tl = nrows - nf * CH
        for j in range(nf):
            pltpu.async_copy(
                rowsv.at[ZB], acc_spm.at[pl.ds(r0 + j * CH, CH)], sem_s.at[ZB]
            )
        if tl:
            pltpu.async_copy(
                rowsv.at[ZB, pl.ds(0, tl)],
                acc_spm.at[pl.ds(r0 + nf * CH, tl)], sem_s.at[ZB],
            )
        for j in range(nf):
            pltpu.make_async_copy(
                rowsv.at[ZB], acc_spm.at[pl.ds(r0 + j * CH, CH)], sem_s.at[ZB]
            ).wait()
        if tl:
            pltpu.make_async_copy(
                rowsv.at[ZB, pl.ds(0, tl)],
                acc_spm.at[pl.ds(r0 + nf * CH, tl)], sem_s.at[ZB],
            ).wait()

    @pl.when(s < NS - 1)
    def _():
        _zero_rows(ROW_CHUNK)

    @pl.when(s == NS - 1)
    def _():
        _zero_rows(ROW_LAST)

    plsc.subcore_barrier()

    for h in range(CPT // HALF):
        g0 = h * HALF

        if h > 0:
            _fetch_idx(g0)
            _wait_idx(g0)

            for j in range(PF):
                @pl.when(g0 + j < nct)
                def _(j=j):
                    _gather(j, j % NBUF)

        def step(k2, _):
            for b in range(NBUF):
                kk = NBUF * k2 + b
                kg = g0 + kk
                bf = (b + PF) % NBUF

                @pl.when(kg < nct)
                def _():
                    _wait_gather(kk, b)
                    _scatter(kk, b)

                @pl.when((kk >= 1) & (kg - 1 < nct))
                def _():
                    _wait_scatter(kk - 1, bf)

                @pl.when((kk + PF <= HALF - 1) & (kg + PF < nct))
                def _():
                    _gather(kk + PF, bf)

            return 0

        lax.fori_loop(0, HALF // NBUF, step, 0)

        @pl.when(g0 + HALF - 1 < nct)
        def _():
            _wait_scatter(HALF - 1, (HALF - 1) % NBUF)

    plsc.subcore_barrier()

    def _readout(nrows):
        nf = nrows // CH
        tl = nrows - nf * CH
        for j in range(nf):
            pltpu.async_copy(
                acc_spm.at[pl.ds(r0 + j * CH, CH)],
                out_hbm.at[pl.ds(c * N + r0 + j * CH, CH)], sem_i,
            )
        if tl:
            pltpu.async_copy(
                acc_spm.at[pl.ds(r0 + nf * CH, tl)],
                out_hbm.at[pl.ds(c * N + r0 + nf * CH, tl)], sem_i,
            )
        for j in range(nf):
            pltpu.make_async_copy(
                acc_spm.at[pl.ds(r0 + j * CH, CH)],
                out_hbm.at[pl.ds(c * N + r0 + j * CH, CH)], sem_i,
            ).wait()
        if tl:
            pltpu.make_async_copy(
                acc_spm.at[pl.ds(r0 + nf * CH, tl)],
                out_hbm.at[pl.ds(c * N + r0 + nf * CH, tl)], sem_i,
            ).wait()

    @pl.when(s < NS - 1)
    def _():
        _readout(ROW_CHUNK)

    @pl.when(s == NS - 1)
    def _():
        _readout(ROW_LAST)


def _sc_aggregate(hs, row2, col2):
    return pl.kernel(
        _aggregate_body,
        out_type=jax.ShapeDtypeStruct((NC * N, D), jnp.float32),
        mesh=_sc_mesh(),
        scratch_types=[
            pltpu.VMEM((HALF, CH), jnp.int32),
            pltpu.VMEM((HALF, CH), jnp.int32),
            pltpu.VMEM((NBUF, CH, D), jnp.float32),
            pltpu.SemaphoreType.DMA((NBUF,)),
            pltpu.SemaphoreType.DMA((NBUF,)),
            pltpu.SemaphoreType.DMA,
            pltpu.VMEM_SHARED((N, D), jnp.float32),
        ],
    )(hs, row2, col2)


NB = 1000


def _tc1_body(x_ref, w_ref, g_ref, rv_ref, d0_ref, d1_ref, hs_ref, dinv_ref):
    deg = d0_ref[...] + d1_ref[...] + 1.0
    dinv = lax.rsqrt(deg)
    a = g_ref[...] * lax.rsqrt(rv_ref[...] + EPS)
    wa = w_ref[...] * a[:, None]
    u = lax.dot_general(
        x_ref[...], wa, (((1,), (1,)), ((), ())),
        preferred_element_type=jnp.float32,
    )
    hs_ref[...] = u * dinv
    dinv_ref[...] = dinv


def _tc1(x, W1, g1, rv1, deg0, deg1):
    rows = lambda i: (i, 0)
    full = lambda i: (0, 0)
    return pl.pallas_call(
        _tc1_body,
        grid=(N // NB,),
        in_specs=[
            pl.BlockSpec((NB, D), rows),
            pl.BlockSpec((D, D), full),
            pl.BlockSpec((D,), lambda i: (0,)),
            pl.BlockSpec((D,), lambda i: (0,)),
            pl.BlockSpec((NB, 1), rows),
            pl.BlockSpec((NB, 1), rows),
        ],
        out_specs=[
            pl.BlockSpec((NB, D), rows),
            pl.BlockSpec((NB, 1), rows),
        ],
        out_shape=[
            jax.ShapeDtypeStruct((N, D), jnp.float32),
            jax.ShapeDtypeStruct((N, 1), jnp.float32),
        ],
    )(x, W1, g1, rv1, deg0, deg1)


def _tc2_body(acc0_ref, acc1_ref, hs1_ref, x_ref, dinv_ref, w2_ref, g2_ref,
              rv2_ref, b1_ref, g1_ref, be1_ref, rm1_ref, rv1_ref,
              h1_ref, hs2_ref):
    dinv = dinv_ref[...]
    a1 = g1_ref[...] * lax.rsqrt(rv1_ref[...] + EPS)
    c1 = be1_ref[...] + (b1_ref[...] - rm1_ref[...]) * a1
    agg = dinv * (acc0_ref[...] + acc1_ref[...] + hs1_ref[...])
    h1 = jnp.maximum(agg + c1[None, :], 0.0) + x_ref[...]
    a2 = g2_ref[...] * lax.rsqrt(rv2_ref[...] + EPS)
    wa2 = w2_ref[...] * a2[:, None]
    u2 = lax.dot_general(
        h1, wa2, (((1,), (1,)), ((), ())), preferred_element_type=jnp.float32
    )
    h1_ref[...] = h1
    hs2_ref[...] = u2 * dinv


def _tc2(acc0, acc1, hs1, x, dinv, W2, g2, rv2, b1, g1, be1, rm1, rv1):
    rows = lambda i: (i, 0)
    full = lambda i: (0, 0)
    vec = lambda i: (0,)
    return pl.pallas_call(
        _tc2_body,
        grid=(N // NB,),
        in_specs=[
            pl.BlockSpec((NB, D), rows),
            pl.BlockSpec((NB, D), rows),
            pl.BlockSpec((NB, D), rows),
            pl.BlockSpec((NB, D), rows),
            pl.BlockSpec((NB, 1), rows),
            pl.BlockSpec((D, D), full),
            pl.BlockSpec((D,), vec),
            pl.BlockSpec((D,), vec),
            pl.BlockSpec((D,), vec),
            pl.BlockSpec((D,), vec),
            pl.BlockSpec((D,), vec),
            pl.BlockSpec((D,), vec),
            pl.BlockSpec((D,), vec),
        ],
        out_specs=[
            pl.BlockSpec((NB, D), rows),
            pl.BlockSpec((NB, D), rows),
        ],
        out_shape=[
            jax.ShapeDtypeStruct((N, D), jnp.float32),
            jax.ShapeDtypeStruct((N, D), jnp.float32),
        ],
    )(acc0, acc1, hs1, x, dinv, W2, g2, rv2, b1, g1, be1, rm1, rv1)


def _tc3_body(acc0_ref, acc1_ref, hs2_ref, h1_ref, dinv_ref, b2_ref, g2_ref,
              be2_ref, rm2_ref, rv2_ref, aw_ref, ab_ref, out_ref):
    dinv = dinv_ref[...]
    a2 = g2_ref[...] * lax.rsqrt(rv2_ref[...] + EPS)
    c2 = be2_ref[...] + (b2_ref[...] - rm2_ref[...]) * a2
    agg = dinv * (acc0_ref[...] + acc1_ref[...] + hs2_ref[...])
    h2 = jnp.maximum(agg + c2[None, :], 0.0) + h1_ref[...]
    t = jnp.sum(h2 * aw_ref[...], axis=1, keepdims=True) + ab_ref[0]
    att = jnp.tanh(t)
    m = jnp.max(att)
    e = jnp.exp(att - m)
    w = e / jnp.sum(e)
    out_ref[...] = h2 * w


def _tc3(acc0, acc1, hs2, h1, dinv, b2, g2, be2, rm2, rv2, att_w, att_b):
    return pl.pallas_call(
        _tc3_body,
        out_shape=jax.ShapeDtypeStruct((N, D), jnp.float32),
    )(acc0, acc1, hs2, h1, dinv, b2, g2, be2, rm2, rv2, att_w, att_b)


def kernel(x, W1, b1, g1, be1, rm1, rv1, W2, b2, g2, be2, rm2, rv2,
           att_w, att_b, edge_index_0, edge_index_1):
    row = jnp.concatenate([edge_index_0[0], edge_index_1[0]])
    col = jnp.concatenate([edge_index_0[1], edge_index_1[1]])
    pad = ((0, 0), (0, CPAD - NCHUNK), (0, 0))
    row2 = jnp.pad(row.reshape(NC, NCHUNK, CH), pad).reshape(NC * CPAD, CH)
    col2 = jnp.pad(col.reshape(NC, NCHUNK, CH), pad).reshape(NC * CPAD, CH)

    degp = _sc_degree(col2)
    hs1, dinv = _tc1(x, W1, g1, rv1, degp[:N, None], degp[N:, None])
    a1p = _sc_aggregate(hs1, row2, col2)
    h1, hs2 = _tc2(a1p[:N], a1p[N:], hs1, x, dinv, W2, g2, rv2,
                   b1, g1, be1, rm1, rv1)
    a2p = _sc_aggregate(hs2, row2, col2)
    out = _tc3(a2p[:N], a2p[N:], hs2, h1, dinv, b2, g2, be2, rm2, rv2,
               att_w, att_b)
    return out


# speedup vs baseline: 1.1005x; 1.1005x over previous
import jax
import jax.numpy as jnp
from jax import lax
from jax.experimental import pallas as pl
from jax.experimental.pallas import tpu as pltpu
from jax.experimental.pallas import tpu_sc as plsc

N = 10000
D = 128
EPS = 1e-5
NC = 2
NS = 16
CH = 64

NCHUNK = 2500
CPAD = 2560
CPT = 160
LAST_CT = NCHUNK - (NS - 1) * CPT
HALF = CPT // 4
NBUF = 4
PF = 3

ROW_CHUNK = 632
ROW_LAST = N - (NS - 1) * ROW_CHUNK
ELEM_CHUNK = 640
ELEM_LAST = N - (NS - 1) * ELEM_CHUNK


def _sc_mesh():
    return plsc.VectorSubcoreMesh(
        core_axis_name="c", subcore_axis_name="s", num_cores=NC, num_subcores=NS
    )


def _degree_body(col_hbm, out_hbm, idxc_all, onesv, bufv, sem, deg_spm):
    c = lax.axis_index("c")
    s = lax.axis_index("s")
    nct = jnp.where(s == NS - 1, LAST_CT, CPT)

    def fill(i, _):
        bufv[pl.ds(i * 16, 16)] = jnp.zeros((16,), jnp.float32)
        return 0

    lax.fori_loop(0, ELEM_CHUNK // 16, fill, 0)

    def fill1(i, _):
        onesv[pl.ds(i * 16, 16)] = jnp.ones((16,), jnp.float32)
        return 0

    lax.fori_loop(0, CH // 16, fill1, 0)

    pltpu.sync_copy(col_hbm.at[pl.ds(CPAD * c + CPT * s, CPT)], idxc_all)

    @pl.when(s < NS - 1)
    def _():
        pltpu.sync_copy(bufv, deg_spm.at[pl.ds(s * ELEM_CHUNK, ELEM_CHUNK)])

    @pl.when(s == NS - 1)
    def _():
        pltpu.sync_copy(
            bufv.at[pl.ds(0, ELEM_LAST)],
            deg_spm.at[pl.ds((NS - 1) * ELEM_CHUNK, ELEM_LAST)],
        )

    plsc.subcore_barrier()

    FIRE = 8

    def step(k8, _):
        for j in range(FIRE):
            kk = FIRE * k8 + j

            @pl.when(kk < nct)
            def _():
                pltpu.async_copy(
                    onesv, deg_spm.at[idxc_all.at[kk]], sem, add=True
                )

        for j in range(FIRE):
            kk = FIRE * k8 + j

            @pl.when(kk < nct)
            def _():
                pltpu.make_async_copy(
                    onesv, deg_spm.at[idxc_all.at[kk]], sem
                ).wait()

        return 0

    lax.fori_loop(0, CPT // FIRE, step, 0)
    plsc.subcore_barrier()

    @pl.when(s < NS - 1)
    def _():
        pltpu.sync_copy(deg_spm.at[pl.ds(s * ELEM_CHUNK, ELEM_CHUNK)], bufv)
        pltpu.sync_copy(
            bufv, out_hbm.at[pl.ds(c * N + s * ELEM_CHUNK, ELEM_CHUNK)]
        )

    @pl.when(s == NS - 1)
    def _():
        pltpu.sync_copy(
            deg_spm.at[pl.ds((NS - 1) * ELEM_CHUNK, ELEM_LAST)],
            bufv.at[pl.ds(0, ELEM_LAST)],
        )
        pltpu.sync_copy(
            bufv.at[pl.ds(0, ELEM_LAST)],
            out_hbm.at[pl.ds(c * N + (NS - 1) * ELEM_CHUNK, ELEM_LAST)],
        )


def _sc_degree(col2):
    return pl.kernel(
        _degree_body,
        out_type=jax.ShapeDtypeStruct((NC * N,), jnp.float32),
        mesh=_sc_mesh(),
        scratch_types=[
            pltpu.VMEM((CPT, CH), jnp.int32),
            pltpu.VMEM((CH,), jnp.float32),
            pltpu.VMEM((ELEM_CHUNK,), jnp.float32),
            pltpu.SemaphoreType.DMA,
            pltpu.VMEM_SHARED((N,), jnp.float32),
        ],
    )(col2)


def _aggregate_body(hs_hbm, row_hbm, col_hbm, out_hbm,
                    idxr_h, idxc_h, rowsv, sem_g, sem_s, sem_i, acc_spm):
    c = lax.axis_index("c")
    s = lax.axis_index("s")
    nct = jnp.where(s == NS - 1, LAST_CT, CPT)
    ibase = CPAD * c + CPT * s
    ZB = NBUF - 1

    def _fetch_idx(g0):
        pltpu.async_copy(row_hbm.at[pl.ds(ibase + g0, HALF)], idxr_h, sem_i)
        pltpu.async_copy(col_hbm.at[pl.ds(ibase + g0, HALF)], idxc_h, sem_i)

    def _wait_idx(g0):
        pltpu.make_async_copy(
            row_hbm.at[pl.ds(ibase + g0, HALF)], idxr_h, sem_i
        ).wait()
        pltpu.make_async_copy(
            col_hbm.at[pl.ds(ibase + g0, HALF)], idxc_h, sem_i
        ).wait()

    _fetch_idx(0)

    def fill(i, _):
        rowsv[ZB, i // 8, pl.ds((i % 8) * 16, 16)] = jnp.zeros((16,), jnp.float32)
        return 0

    lax.fori_loop(0, CH * D // 16, fill, 0)

    r0 = s * ROW_CHUNK

    def _gather(kk_local, b):
        pltpu.async_copy(
            hs_hbm.at[idxr_h.at[kk_local]], rowsv.at[b], sem_g.at[b]
        )

    def _wait_gather(kk_local, b):
        pltpu.make_async_copy(
            hs_hbm.at[idxr_h.at[kk_local]], rowsv.at[b], sem_g.at[b]
        ).wait()

    def _scatter(kk_local, b):
        pltpu.async_copy(
            rowsv.at[b], acc_spm.at[idxc_h.at[kk_local]], sem_s.at[b],
            add=True,
        )

    def _wait_scatter(kk_local, b):
        pltpu.make_async_copy(
            rowsv.at[b], acc_spm.at[idxc_h.at[kk_local]], sem_s.at[b]
        ).wait()

    _wait_idx(0)
    for j in range(PF):
        _gather(j, j % NBUF)

    def _zero_rows(nrows):
        nf = nrows // CH
        tl = nrows - nf * CH
        for j in range(nf):
            pltpu.async_copy(
                rowsv.at[ZB], acc_spm.at[pl.ds(r0 + j * CH, CH)], sem_s.at[ZB]
            )
        if tl:
            pltpu.async_copy(
                rowsv.at[ZB, pl.ds(0, tl)],
                acc_spm.at[pl.ds(r0 + nf * CH, tl)], sem_s.at[ZB],
            )
        for j in range(nf):
            pltpu.make_async_copy(
                rowsv.at[ZB], acc_spm.at[pl.ds(r0 + j * CH, CH)], sem_s.at[ZB]
            ).wait()
        if tl:
            pltpu.make_async_copy(
                rowsv.at[ZB, pl.ds(0, tl)],
                acc_spm.at[pl.ds(r0 + nf * CH, tl)], sem_s.at[ZB],
            ).wait()

    @pl.when(s < NS - 1)
    def _():
        _zero_rows(ROW_CHUNK)

    @pl.when(s == NS - 1)
    def _():
        _zero_rows(ROW_LAST)

    plsc.subcore_barrier()

    for h in range(CPT // HALF):
        g0 = h * HALF

        if h > 0:
            _fetch_idx(g0)
            _wait_idx(g0)

            for j in range(PF):
                @pl.when(g0 + j < nct)
                def _(j=j):
                    _gather(j, j % NBUF)

        def step(k2, _):
            for b in range(NBUF):
                kk = NBUF * k2 + b
                kg = g0 + kk
                bf = (b + PF) % NBUF

                @pl.when(kg < nct)
                def _():
                    _wait_gather(kk, b)
                    _scatter(kk, b)

                @pl.when((kk >= 1) & (kg - 1 < nct))
                def _():
                    _wait_scatter(kk - 1, bf)

                @pl.when((kk + PF <= HALF - 1) & (kg + PF < nct))
                def _():
                    _gather(kk + PF, bf)

            return 0

        lax.fori_loop(0, HALF // NBUF, step, 0)

        @pl.when(g0 + HALF - 1 < nct)
        def _():
            _wait_scatter(HALF - 1, (HALF - 1) % NBUF)

    plsc.subcore_barrier()

    def _readout(nrows):
        nf = nrows // CH
        tl = nrows - nf * CH
        for j in range(nf):
            pltpu.async_copy(
                acc_spm.at[pl.ds(r0 + j * CH, CH)],
                out_hbm.at[pl.ds(c * N + r0 + j * CH, CH)], sem_i,
            )
        if tl:
            pltpu.async_copy(
                acc_spm.at[pl.ds(r0 + nf * CH, tl)],
                out_hbm.at[pl.ds(c * N + r0 + nf * CH, tl)], sem_i,
            )
        for j in range(nf):
            pltpu.make_async_copy(
                acc_spm.at[pl.ds(r0 + j * CH, CH)],
                out_hbm.at[pl.ds(c * N + r0 + j * CH, CH)], sem_i,
            ).wait()
        if tl:
            pltpu.make_async_copy(
                acc_spm.at[pl.ds(r0 + nf * CH, tl)],
                out_hbm.at[pl.ds(c * N + r0 + nf * CH, tl)], sem_i,
            ).wait()

    @pl.when(s < NS - 1)
    def _():
        _readout(ROW_CHUNK)

    @pl.when(s == NS - 1)
    def _():
        _readout(ROW_LAST)


def _sc_aggregate(hs, row2, col2):
    return pl.kernel(
        _aggregate_body,
        out_type=jax.ShapeDtypeStruct((NC * N, D), jnp.float32),
        mesh=_sc_mesh(),
        scratch_types=[
            pltpu.VMEM((HALF, CH), jnp.int32),
            pltpu.VMEM((HALF, CH), jnp.int32),
            pltpu.VMEM((NBUF, CH, D), jnp.float32),
            pltpu.SemaphoreType.DMA((NBUF,)),
            pltpu.SemaphoreType.DMA((NBUF,)),
            pltpu.SemaphoreType.DMA,
            pltpu.VMEM_SHARED((N, D), jnp.float32),
        ],
    )(hs, row2, col2)


def _tc1_body(x_ref, w_ref, g_ref, rv_ref, degp_ref, hs_ref, dinv_ref):
    deg2 = degp_ref[...]
    deg = deg2[0] + deg2[1] + 1.0
    dinv = lax.rsqrt(deg)
    a = g_ref[...] * lax.rsqrt(rv_ref[...] + EPS)
    wa = w_ref[...] * a[:, None]
    u = lax.dot_general(
        x_ref[...], wa, (((1,), (1,)), ((), ())),
        preferred_element_type=jnp.float32,
    )
    hs_ref[...] = u * dinv[:, None]
    dinv_ref[...] = dinv[:, None]


def _tc1(x, W1, g1, rv1, degp):
    return pl.pallas_call(
        _tc1_body,
        out_shape=[
            jax.ShapeDtypeStruct((N, D), jnp.float32),
            jax.ShapeDtypeStruct((N, 1), jnp.float32),
        ],
    )(x, W1, g1, rv1, degp)


def _tc2_body(accp_ref, hs1_ref, x_ref, dinv_ref, w2_ref, g2_ref, rv2_ref,
              b1_ref, g1_ref, be1_ref, rm1_ref, rv1_ref, h1_ref, hs2_ref):
    dinv = dinv_ref[...]
    a1 = g1_ref[...] * lax.rsqrt(rv1_ref[...] + EPS)
    c1 = be1_ref[...] + (b1_ref[...] - rm1_ref[...]) * a1
    accp = accp_ref[...]
    agg = dinv * (accp[0] + accp[1] + hs1_ref[...])
    h1 = jnp.maximum(agg + c1[None, :], 0.0) + x_ref[...]
    a2 = g2_ref[...] * lax.rsqrt(rv2_ref[...] + EPS)
    wa2 = w2_ref[...] * a2[:, None]
    u2 = lax.dot_general(
        h1, wa2, (((1,), (1,)), ((), ())), preferred_element_type=jnp.float32
    )
    h1_ref[...] = h1
    hs2_ref[...] = u2 * dinv


def _tc2(accp, hs1, x, dinv, W2, g2, rv2, b1, g1, be1, rm1, rv1):
    return pl.pallas_call(
        _tc2_body,
        out_shape=[
            jax.ShapeDtypeStruct((N, D), jnp.float32),
            jax.ShapeDtypeStruct((N, D), jnp.float32),
        ],
    )(accp, hs1, x, dinv, W2, g2, rv2, b1, g1, be1, rm1, rv1)


def _tc3_body(accp_ref, hs2_ref, h1_ref, dinv_ref, b2_ref, g2_ref, be2_ref,
              rm2_ref, rv2_ref, aw_ref, ab_ref, out_ref):
    dinv = dinv_ref[...]
    a2 = g2_ref[...] * lax.rsqrt(rv2_ref[...] + EPS)
    c2 = be2_ref[...] + (b2_ref[...] - rm2_ref[...]) * a2
    accp = accp_ref[...]
    agg = dinv * (accp[0] + accp[1] + hs2_ref[...])
    h2 = jnp.maximum(agg + c2[None, :], 0.0) + h1_ref[...]
    t = jnp.sum(h2 * aw_ref[...], axis=1, keepdims=True) + ab_ref[0]
    att = jnp.tanh(t)
    m = jnp.max(att)
    e = jnp.exp(att - m)
    w = e / jnp.sum(e)
    out_ref[...] = h2 * w


def _tc3(accp, hs2, h1, dinv, b2, g2, be2, rm2, rv2, att_w, att_b):
    return pl.pallas_call(
        _tc3_body,
        out_shape=jax.ShapeDtypeStruct((N, D), jnp.float32),
    )(accp, hs2, h1, dinv, b2, g2, be2, rm2, rv2, att_w, att_b)


def kernel(x, W1, b1, g1, be1, rm1, rv1, W2, b2, g2, be2, rm2, rv2,
           att_w, att_b, edge_index_0, edge_index_1):
    row = jnp.concatenate([edge_index_0[0], edge_index_1[0]])
    col = jnp.concatenate([edge_index_0[1], edge_index_1[1]])
    pad = ((0, 0), (0, CPAD - NCHUNK), (0, 0))
    row2 = jnp.pad(row.reshape(NC, NCHUNK, CH), pad).reshape(NC * CPAD, CH)
    col2 = jnp.pad(col.reshape(NC, NCHUNK, CH), pad).reshape(NC * CPAD, CH)

    degp = _sc_degree(col2).reshape(NC, N)
    hs1, dinv = _tc1(x, W1, g1, rv1, degp)
    acc1 = _sc_aggregate(hs1, row2, col2).reshape(NC, N, D)
    h1, hs2 = _tc2(acc1, hs1, x, dinv, W2, g2, rv2, b1, g1, be1, rm1, rv1)
    acc2 = _sc_aggregate(hs2, row2, col2).reshape(NC, N, D)
    out = _tc3(acc2, hs2, h1, dinv, b2, g2, be2, rm2, rv2, att_w, att_b)
    return out
